# Initial kernel scaffold; baseline (speedup 1.0000x reference)
import jax
import jax.numpy as jnp
from jax.experimental import pallas as pl


def kernel(value, value_spatial_shapes, sampling_locations, attention_weights):
    raise NotImplementedError("write your pallas kernel here")


# speedup vs baseline: 56.5979x; 56.5979x over previous
import functools

import jax
import jax.numpy as jnp
from jax import lax
from jax.experimental import pallas as pl
from jax.experimental.pallas import tpu as pltpu
from jax.experimental.pallas import tpu_sc as plsc

NC = 2
NS = 16
L = 16

H_SP = 64
W_SP = 64
HEADS = 8
D = 32
Q = 4096
P = 4
PAIRS = 32
NWORD = D // 2
STRIDE = NWORD + 1
R = 128
GROUPS = R // L
NCHUNK = Q // R


def _floor_i32(v):
    t = v.astype(jnp.int32)
    tf = t.astype(jnp.float32)
    return jnp.where(v < tf, t - 1, t)


def _body(slab_hbm, loc_hbm, aw_hbm, out_hbm, slab_v, loc_v, aw_v, out_v):
    wid = lax.axis_index("c") * NS + lax.axis_index("s")
    pltpu.sync_copy(slab_hbm.at[wid], slab_v)
    lanes = lax.iota(jnp.int32, L)

    def chunk_body(c, carry):
        pltpu.sync_copy(loc_hbm.at[wid, pl.ds(c * (R * 2 * P), R * 2 * P)], loc_v)
        pltpu.sync_copy(aw_hbm.at[wid, pl.ds(c * (R * P), R * P)], aw_v)

        def group_body(g, gcarry):
            base8 = g * (L * 2 * P)
            base4 = g * (L * P)
            wbfs = []
            tbases = []
            for p in range(P):
                xx = plsc.load_gather(loc_v, [base8 + lanes * (2 * P) + 2 * p])
                yy = plsc.load_gather(loc_v, [base8 + lanes * (2 * P) + 2 * p + 1])
                aa = plsc.load_gather(aw_v, [base4 + lanes * P + p])
                ix = xx * jnp.float32(W_SP) - 0.5
                iy = yy * jnp.float32(H_SP) - 0.5
                x0 = _floor_i32(ix)
                y0 = _floor_i32(iy)
                wx1 = ix - x0.astype(jnp.float32)
                wx0 = 1.0 - wx1
                wy1 = iy - y0.astype(jnp.float32)
                wy0 = 1.0 - wy1
                for sy in (0, 1):
                    yc = y0 + sy
                    wy = wy1 if sy else wy0
                    vy = (yc >= 0) & (yc <= H_SP - 1)
                    ycc = jnp.clip(yc, 0, H_SP - 1)
                    for sx in (0, 1):
                        xc = x0 + sx
                        wx = wx1 if sx else wx0
                        ok = vy & (xc >= 0) & (xc <= W_SP - 1)
                        xcc = jnp.clip(xc, 0, W_SP - 1)
                        t = ycc * W_SP + xcc
                        w = jnp.where(ok, aa * wx * wy, 0.0)
                        wbfs.append(
                            plsc.pack(w, w, format=plsc.PackFormat.INTERLEAVED)
                        )
                        tbases.append(t * STRIDE)
            for wp in range(NWORD):
                acc = jnp.zeros((2 * L,), jnp.bfloat16)
                for s in range(16):
                    word = plsc.load_gather(slab_v, [tbases[s] + wp])
                    acc = acc + plsc.bitcast(word, jnp.bfloat16) * wbfs[s]
                out_v[wp, pl.ds(g * L, L)] = plsc.bitcast(acc, jnp.int32)
            return gcarry

        lax.fori_loop(0, GROUPS, group_body, 0)
        pltpu.sync_copy(out_v, out_hbm.at[wid, c])
        return carry

    lax.fori_loop(0, NCHUNK, chunk_body, 0)


@jax.jit
def _run(slab, loc_t, aw_t):
    kfn = pl.kernel(
        _body,
        out_type=jax.ShapeDtypeStruct((PAIRS, NCHUNK, NWORD, R), jnp.int32),
        mesh=plsc.VectorSubcoreMesh(
            core_axis_name="c", subcore_axis_name="s",
            num_cores=NC, num_subcores=NS,
        ),
        scratch_types=[
            pltpu.VMEM((H_SP * W_SP * STRIDE,), jnp.int32),
            pltpu.VMEM((R * 2 * P,), jnp.float32),
            pltpu.VMEM((R * P,), jnp.float32),
            pltpu.VMEM((NWORD, R), jnp.int32),
        ],
        compiler_params=pltpu.CompilerParams(needs_layout_passes=False),
    )
    return kfn(slab, loc_t, aw_t)


def kernel(value, value_spatial_shapes, sampling_locations, attention_weights):
    bs, K, heads, d = value.shape
    v_bf = value.astype(jnp.bfloat16)
    v_p = v_bf.transpose(0, 2, 1, 3).reshape(PAIRS, K, NWORD, 2)
    slab = lax.bitcast_convert_type(v_p, jnp.int32)
    slab = jnp.concatenate(
        [slab, jnp.zeros((PAIRS, K, 1), jnp.int32)], axis=-1
    ).reshape(PAIRS, K * STRIDE)

    loc_t = (
        sampling_locations[:, :, :, 0]
        .transpose(0, 2, 1, 3, 4)
        .reshape(PAIRS, Q * P * 2)
    )
    aw_t = (
        attention_weights[:, :, :, 0]
        .transpose(0, 2, 1, 3)
        .reshape(PAIRS, Q * P)
    )

    out_i32 = _run(slab, loc_t, aw_t)
    out_bf = lax.bitcast_convert_type(out_i32, jnp.bfloat16)
    out = (
        out_bf.transpose(0, 1, 3, 2, 4)
        .reshape(bs, heads, Q, d)
        .transpose(0, 2, 1, 3)
        .reshape(bs, Q, heads * d)
        .astype(jnp.float32)
    )
    return out
